# baseline (speedup 1.0000x reference)
import functools

import jax
import jax.numpy as jnp
from jax import lax
from jax.experimental import pallas as pl
from jax.experimental.pallas import tpu as pltpu
from jax.experimental.pallas import tpu_sc as plsc

VOCAB = 1000000
EMBED_DIM = 64
HIDDEN = 256
NUM_CLASSES = 50
BATCH = 4096
SEQ = 200

NC = 2
NS = 16
NW = NC * NS
BPW = BATCH // NW
CHUNK = 104
CPB = 2
PAD = CPB * CHUNK - SEQ
NCH = BPW * CPB
INV_SEQ = 1.0 / SEQ


def _pool_body(x_hbm, emb_hbm, out_hbm, idx_v, rows_a, rows_b, pooled_v,
               sem_a, sem_b):
    wid = lax.axis_index("s") * NC + lax.axis_index("c")
    pltpu.sync_copy(x_hbm.at[pl.ds(wid * NCH, NCH)], idx_v)

    pltpu.async_copy(emb_hbm.at[idx_v.at[0]], rows_a, sem_a)
    pltpu.async_copy(emb_hbm.at[idx_v.at[1]], rows_b, sem_b)

    def accum(rows, init, lo):
        def j_body(j, acc):
            return tuple(
                acc[i] + rows[j, pl.ds(16 * i, 16)] for i in range(4))
        return lax.fori_loop(lo, CHUNK, j_body, init, unroll=8)

    def b_body(b, _):
        pltpu.make_async_copy(emb_hbm.at[idx_v.at[0]], rows_a, sem_a).wait()
        corr = tuple(rows_a[0, pl.ds(16 * i, 16)] for i in range(4))
        acc = accum(rows_a, corr, 1)

        @pl.when(b < BPW - 1)
        def _():
            pltpu.async_copy(emb_hbm.at[idx_v.at[2 * b + 2]], rows_a, sem_a)

        pltpu.make_async_copy(emb_hbm.at[idx_v.at[0]], rows_b, sem_b).wait()
        acc = accum(rows_b, acc, 0)

        @pl.when(b < BPW - 1)
        def _():
            pltpu.async_copy(emb_hbm.at[idx_v.at[2 * b + 3]], rows_b, sem_b)

        for i in range(4):
            pooled_v[b, pl.ds(16 * i, 16)] = (
                acc[i] - 8.0 * corr[i]) * INV_SEQ
        return 0

    lax.fori_loop(0, BPW, b_body, 0)
    pltpu.sync_copy(pooled_v, out_hbm.at[pl.ds(wid * BPW, BPW)])


@functools.partial(
    pl.kernel,
    out_type=jax.ShapeDtypeStruct((BATCH, EMBED_DIM), jnp.float32),
    mesh=plsc.VectorSubcoreMesh(core_axis_name="c", subcore_axis_name="s"),
    compiler_params=pltpu.CompilerParams(use_tc_tiling_on_sc=False),
    scratch_types=[
        pltpu.VMEM((NCH, CHUNK), jnp.int32),
        pltpu.VMEM((CHUNK, EMBED_DIM), jnp.float32),
        pltpu.VMEM((CHUNK, EMBED_DIM), jnp.float32),
        pltpu.VMEM((BPW, EMBED_DIM), jnp.float32),
        pltpu.SemaphoreType.DMA,
        pltpu.SemaphoreType.DMA,
    ],
)
def _pool_sc(x_hbm, emb_hbm, out_hbm, idx_v, rows_a, rows_b, pooled_v,
             sem_a, sem_b):
    _pool_body(x_hbm, emb_hbm, out_hbm, idx_v, rows_a, rows_b, pooled_v,
               sem_a, sem_b)


def _mlp_body(p_ref, w1_ref, b1_ref, w2_ref, b2_ref, o_ref):
    h = jnp.dot(p_ref[...], w1_ref[...], preferred_element_type=jnp.float32)
    h = jnp.maximum(h + b1_ref[...], 0.0)
    o_ref[...] = (
        jnp.dot(h, w2_ref[...], preferred_element_type=jnp.float32)
        + b2_ref[...])


def _mlp_tc(pooled, W1, b1, W2p, b2p):
    return pl.pallas_call(
        _mlp_body,
        out_shape=jax.ShapeDtypeStruct((BATCH, 128), jnp.float32),
    )(pooled, W1, b1, W2p, b2p)


@jax.jit
def kernel(x, emb, W1, b1, W2, b2):
    x = x.astype(jnp.int32)
    xp = jnp.concatenate(
        [x, jnp.broadcast_to(x[:, :1], (BATCH, PAD))], axis=1)
    xp = xp.reshape(BATCH * CPB, CHUNK)

    pooled = _pool_sc(xp, emb)

    W2p = jnp.pad(W2, ((0, 0), (0, 128 - NUM_CLASSES)))
    b2p = jnp.pad(b2, (0, 128 - NUM_CLASSES)).reshape(1, 128)
    out = _mlp_tc(pooled, W1, b1.reshape(1, HIDDEN), W2p, b2p)
    return out[:, :NUM_CLASSES]


# speedup vs baseline: 1.0196x; 1.0196x over previous
import functools

import jax
import jax.numpy as jnp
from jax import lax
from jax.experimental import pallas as pl
from jax.experimental.pallas import tpu as pltpu
from jax.experimental.pallas import tpu_sc as plsc

VOCAB = 1000000
EMBED_DIM = 64
HIDDEN = 256
NUM_CLASSES = 50
BATCH = 4096
SEQ = 200

NC = 2
NS = 16
NW = NC * NS
BPW = BATCH // NW
CHUNK_A = 104
CHUNK_B = SEQ - CHUNK_A
INV_SEQ = 1.0 / SEQ


def _pool_body(x_hbm, emb_hbm, out_hbm, idx_v, rows_a, rows_b, pooled_v,
               sem_a, sem_b):
    wid = lax.axis_index("s") * NC + lax.axis_index("c")
    base = wid * BPW
    pltpu.sync_copy(x_hbm.at[pl.ds(base, BPW)], idx_v)

    def start_a(b):
        pltpu.async_copy(
            emb_hbm.at[idx_v.at[b, pl.ds(0, CHUNK_A)]], rows_a, sem_a)

    def start_b(b):
        pltpu.async_copy(
            emb_hbm.at[idx_v.at[b, pl.ds(CHUNK_A, CHUNK_B)]], rows_b, sem_b)

    start_a(0)
    start_b(0)

    def accum(rows, init, lo, hi):
        def j_body(j, acc):
            return tuple(
                acc[i] + rows[j, pl.ds(16 * i, 16)] for i in range(4))
        return lax.fori_loop(lo, hi, j_body, init, unroll=8)

    def b_body(b, _):
        pltpu.make_async_copy(
            emb_hbm.at[idx_v.at[0, pl.ds(0, CHUNK_A)]], rows_a, sem_a).wait()
        acc = tuple(rows_a[0, pl.ds(16 * i, 16)] for i in range(4))
        acc = accum(rows_a, acc, 1, CHUNK_A)

        @pl.when(b < BPW - 1)
        def _():
            start_a(b + 1)

        pltpu.make_async_copy(
            emb_hbm.at[idx_v.at[0, pl.ds(CHUNK_A, CHUNK_B)]], rows_b,
            sem_b).wait()
        acc = accum(rows_b, acc, 0, CHUNK_B)

        @pl.when(b < BPW - 1)
        def _():
            start_b(b + 1)

        for i in range(4):
            pooled_v[b, pl.ds(16 * i, 16)] = acc[i] * INV_SEQ
        return 0

    lax.fori_loop(0, BPW, b_body, 0)
    pltpu.sync_copy(pooled_v, out_hbm.at[pl.ds(base, BPW)])


@functools.partial(
    pl.kernel,
    out_type=jax.ShapeDtypeStruct((BATCH, EMBED_DIM), jnp.float32),
    mesh=plsc.VectorSubcoreMesh(core_axis_name="c", subcore_axis_name="s"),
    compiler_params=pltpu.CompilerParams(use_tc_tiling_on_sc=False),
    scratch_types=[
        pltpu.VMEM((BPW, SEQ), jnp.int32),
        pltpu.VMEM((CHUNK_A, EMBED_DIM), jnp.float32),
        pltpu.VMEM((CHUNK_B, EMBED_DIM), jnp.float32),
        pltpu.VMEM((BPW, EMBED_DIM), jnp.float32),
        pltpu.SemaphoreType.DMA,
        pltpu.SemaphoreType.DMA,
    ],
)
def _pool_sc(x_hbm, emb_hbm, out_hbm, idx_v, rows_a, rows_b, pooled_v,
             sem_a, sem_b):
    _pool_body(x_hbm, emb_hbm, out_hbm, idx_v, rows_a, rows_b, pooled_v,
               sem_a, sem_b)


def _mlp_body(p_ref, w1_ref, b1_ref, w2_ref, b2_ref, o_ref):
    h = jnp.dot(p_ref[...], w1_ref[...], preferred_element_type=jnp.float32)
    h = jnp.maximum(h + b1_ref[...], 0.0)
    o_ref[...] = (
        jnp.dot(h, w2_ref[...], preferred_element_type=jnp.float32)
        + b2_ref[...])


def _mlp_tc(pooled, W1, b1, W2p, b2p):
    return pl.pallas_call(
        _mlp_body,
        out_shape=jax.ShapeDtypeStruct((BATCH, 128), jnp.float32),
    )(pooled, W1, b1, W2p, b2p)


@jax.jit
def kernel(x, emb, W1, b1, W2, b2):
    x = x.astype(jnp.int32)
    pooled = _pool_sc(x, emb)

    W2p = jnp.pad(W2, ((0, 0), (0, 128 - NUM_CLASSES)))
    b2p = jnp.pad(b2, (0, 128 - NUM_CLASSES)).reshape(1, 128)
    out = _mlp_tc(pooled, W1, b1.reshape(1, HIDDEN), W2p, b2p)
    return out[:, :NUM_CLASSES]
